# Initial kernel scaffold; baseline (speedup 1.0000x reference)
import jax
import jax.numpy as jnp
from jax.experimental import pallas as pl


def kernel(x, edge_index, W, b):
    raise NotImplementedError("write your pallas kernel here")


# speedup vs baseline: 7.1580x; 7.1580x over previous
import functools

import jax
import jax.numpy as jnp
from jax import lax
from jax.experimental import pallas as pl
from jax.experimental.pallas import tpu as pltpu
from jax.experimental.pallas import tpu_sc as plsc

NC = 2
NS = 16
NW = NC * NS
BLK = 128


def _mm_body(x_ref, w_ref, o_ref):
    o_ref[...] = jnp.dot(x_ref[...], w_ref[...],
                         preferred_element_type=jnp.float32)


def _matmul(x, W, block_rows):
    n, d = x.shape
    c = W.shape[1]
    return pl.pallas_call(
        _mm_body,
        grid=(n // block_rows,),
        in_specs=[
            pl.BlockSpec((block_rows, d), lambda i: (i, 0)),
            pl.BlockSpec((d, c), lambda i: (0, 0)),
        ],
        out_specs=pl.BlockSpec((block_rows, c), lambda i: (i, 0)),
        out_shape=jax.ShapeDtypeStruct((n, c), jnp.float32),
    )(x, W)


def _comb_body(y_ref, p0_ref, p1_ref, b_ref, o_ref):
    o_ref[...] = y_ref[...] + p0_ref[...] + p1_ref[...] + b_ref[...]


def _combine(y, p0, p1, b, block_rows):
    n, c = y.shape
    b2 = b.reshape(1, c)
    return pl.pallas_call(
        _comb_body,
        grid=(n // block_rows,),
        in_specs=[
            pl.BlockSpec((block_rows, c), lambda i: (i, 0)),
            pl.BlockSpec((block_rows, c), lambda i: (i, 0)),
            pl.BlockSpec((block_rows, c), lambda i: (i, 0)),
            pl.BlockSpec((1, c), lambda i: (0, 0)),
        ],
        out_specs=pl.BlockSpec((block_rows, c), lambda i: (i, 0)),
        out_shape=jax.ShapeDtypeStruct((n, c), jnp.float32),
    )(y, p0, p1, b2)


def _make_sc_scatter(n_rows, c, nblk, acc_rows):
    rows_per_tile = acc_rows // NS
    zrows = 16

    def body(src_hbm, dst_hbm, y_hbm, out_hbm,
             src_v, dst_v, rows_v, zbuf, acc, sem):
        cid = lax.axis_index("c")
        sid = lax.axis_index("s")
        wid = sid * NC + cid

        for i in range(zrows):
            for j in range(c // 16):
                zbuf[i, pl.ds(j * 16, 16)] = jnp.zeros((16,), jnp.float32)
        base = sid * rows_per_tile

        def zero_step(k, _):
            pltpu.sync_copy(zbuf, acc.at[pl.ds(base + k * zrows, zrows)])
            return 0
        lax.fori_loop(0, rows_per_tile // zrows, zero_step, 0)
        plsc.subcore_barrier()

        pltpu.sync_copy(src_hbm.at[wid], src_v)
        pltpu.sync_copy(dst_hbm.at[wid], dst_v)

        def edge_step(j, _):
            pltpu.async_copy(y_hbm.at[src_v.at[j]], rows_v, sem).wait()
            pltpu.sync_copy(rows_v, acc.at[dst_v.at[j]], add=True)
            return 0
        lax.fori_loop(0, nblk, edge_step, 0)
        plsc.subcore_barrier()

        pltpu.sync_copy(acc.at[pl.ds(base, rows_per_tile)],
                        out_hbm.at[cid, pl.ds(base, rows_per_tile)])

    mesh = plsc.VectorSubcoreMesh(core_axis_name="c", subcore_axis_name="s")
    return pl.kernel(
        body,
        out_type=jax.ShapeDtypeStruct((NC, acc_rows, c), jnp.float32),
        mesh=mesh,
        compiler_params=pltpu.CompilerParams(use_tc_tiling_on_sc=False),
        scratch_types=[
            pltpu.VMEM((nblk, BLK), jnp.int32),
            pltpu.VMEM((nblk, BLK), jnp.int32),
            pltpu.VMEM((BLK, c), jnp.float32),
            pltpu.VMEM((zrows, c), jnp.float32),
            pltpu.VMEM_SHARED((acc_rows, c), jnp.float32),
            pltpu.SemaphoreType.DMA,
        ],
    )


@jax.jit
def kernel(x, edge_index, W, b):
    n, d = x.shape
    c = W.shape[1]
    e = edge_index.shape[1]

    per_tile = -(-e // NW)
    nblk = -(-per_tile // BLK)
    ep = NW * nblk * BLK
    acc_rows = -(-n // (16 * NS)) * (16 * NS)
    if acc_rows == n:
        acc_rows += 16 * NS

    pad = ep - e
    src = jnp.concatenate([edge_index[0], jnp.zeros((pad,), jnp.int32)])
    dst = jnp.concatenate([edge_index[1], jnp.full((pad,), n, jnp.int32)])
    src_r = src.reshape(NW, nblk, BLK)
    dst_r = dst.reshape(NW, nblk, BLK)

    y = _matmul(x, W, block_rows=1000)
    partials = _make_sc_scatter(n, c, nblk, acc_rows)(src_r, dst_r, y)
    out = _combine(y, partials[0, :n], partials[1, :n], b, block_rows=1000)
    return out
